# baseline (speedup 1.0000x reference)
import functools

import jax
import jax.numpy as jnp
from jax import lax
from jax.experimental import pallas as pl
from jax.experimental.pallas import tpu as pltpu
from jax.experimental.pallas import tpu_sc as plsc

N_ATOMS = 100000
EMB = 128
TABLE_ROWS = 101
CHUNK = 128
UNIT = 2 * CHUNK

_info = plsc.get_sparse_core_info()
NC = _info.num_cores
NS = _info.num_subcores
NW = NC * NS

CHUNKS_PER_W = -(-N_ATOMS // (CHUNK * NW))
SPAN = CHUNKS_PER_W * CHUNK
UNITS = SPAN // UNIT
PAIRS = UNITS // 2


def _make_lookup():
    mesh = plsc.VectorSubcoreMesh(core_axis_name="c", subcore_axis_name="s")

    @functools.partial(
        pl.kernel,
        mesh=mesh,
        out_type=jax.ShapeDtypeStruct((N_ATOMS, EMB), jnp.float32),
        scratch_types=[
            pltpu.VMEM((SPAN,), jnp.int32),
            pltpu.VMEM((UNIT, EMB), jnp.float32),
            pltpu.VMEM((UNIT, EMB), jnp.float32),
            pltpu.VMEM_SHARED((TABLE_ROWS, EMB), jnp.float32),
            pltpu.SemaphoreType.DMA,
            pltpu.SemaphoreType.DMA,
            pltpu.SemaphoreType.DMA,
            pltpu.SemaphoreType.DMA,
        ],
    )
    def lookup(z_hbm, table_hbm, out_hbm, idx_all, rows0, rows1,
               table_sh, gsem0, gsem1, wsem0, wsem1):
        sid = lax.axis_index("s")
        wid = sid * NC + lax.axis_index("c")

        @pl.when(sid == 0)
        def _():
            pltpu.sync_copy(table_hbm, table_sh.at[pl.ds(1, TABLE_ROWS - 1)])

        start = jnp.minimum(wid * SPAN, N_ATOMS - SPAN)
        pltpu.sync_copy(z_hbm.at[pl.ds(start, SPAN)], idx_all)

        plsc.subcore_barrier()

        rows = (rows0, rows1)
        gsem = (gsem0, gsem1)
        wsem = (wsem0, wsem1)

        def issue_unit_gathers(u, b):
            for h in range(2):
                pltpu.async_copy(
                    table_sh.at[idx_all.at[pl.ds(u * UNIT + h * CHUNK, CHUNK)]],
                    rows[b].at[pl.ds(h * CHUNK, CHUNK)], gsem[b])

        def issue_tail_gather(b):
            pltpu.async_copy(
                table_sh.at[idx_all.at[pl.ds(UNITS * UNIT, CHUNK)]],
                rows[b].at[pl.ds(0, CHUNK)], gsem[b])

        def drain_full(sem, b):
            pltpu.make_async_copy(out_hbm.at[pl.ds(0, UNIT)], rows[b],
                                  sem).wait()

        def drain_half(sem, b):
            pltpu.make_async_copy(out_hbm.at[pl.ds(0, CHUNK)],
                                  rows[b].at[pl.ds(0, CHUNK)], sem).wait()

        issue_unit_gathers(0, 0)

        def pair_body(p, carry):
            for b in range(2):
                u = 2 * p + b
                if b == 0:
                    @pl.when(p > 0)
                    def _():
                        drain_full(wsem[1], 1)
                else:
                    drain_full(wsem[0], 0)
                if b == 0:
                    issue_unit_gathers(u + 1, 1)
                else:
                    @pl.when(p < PAIRS - 1)
                    def _():
                        issue_unit_gathers(u + 1, 0)

                    @pl.when(p == PAIRS - 1)
                    def _():
                        issue_tail_gather(0)
                drain_full(gsem[b], b)
                pltpu.async_copy(rows[b],
                                 out_hbm.at[pl.ds(start + u * UNIT, UNIT)],
                                 wsem[b])
            return carry

        lax.fori_loop(0, PAIRS, pair_body, 0)

        drain_full(wsem[1], 1)
        drain_half(gsem[0], 0)
        pltpu.sync_copy(rows0.at[pl.ds(0, CHUNK)],
                        out_hbm.at[pl.ds(start + UNITS * UNIT, CHUNK)])

    return lookup


_lookup = _make_lookup()


def kernel(Z, W):
    return _lookup(Z, W)
